# baseline (speedup 1.0000x reference)
import functools

import jax
import jax.numpy as jnp
from jax import lax
from jax.experimental import pallas as pl
from jax.experimental.pallas import tpu as pltpu
from jax.experimental.pallas import tpu_sc as plsc

HIDDEN = 128
B, T = 4, 2048
EPS = 1e-6
NTOK = B * T
NW = 32
ROWS_PER_W = NTOK // NW
WPB = T // ROWS_PER_W
L = 16
NCH = HIDDEN // L
GCH = 64
NG = ROWS_PER_W // GCH


def _rsqrt16(x):
    i = lax.bitcast_convert_type(x, jnp.int32)
    i = jnp.int32(0x5F3759DF) - lax.shift_right_arithmetic(i, 1)
    y = lax.bitcast_convert_type(i, jnp.float32)
    xh = x * 0.5
    for _ in range(3):
        y = y * (1.5 - xh * y * y)
    return y


@functools.partial(
    pl.kernel,
    out_type=jax.ShapeDtypeStruct((B, T, HIDDEN), jnp.float32),
    mesh=plsc.VectorSubcoreMesh(core_axis_name="c", subcore_axis_name="s"),
    scratch_types=[
        pltpu.VMEM((ROWS_PER_W,), jnp.int32),
        pltpu.VMEM((ROWS_PER_W,), jnp.int32),
        pltpu.VMEM((2, HIDDEN), jnp.float32),
        pltpu.VMEM((HIDDEN,), jnp.float32),
        pltpu.VMEM((ROWS_PER_W, HIDDEN), jnp.float32),
        pltpu.SemaphoreType.DMA,
        pltpu.SemaphoreType.DMA,
        pltpu.SemaphoreType.DMA,
        pltpu.SemaphoreType.DMA,
        pltpu.SemaphoreType.DMA,
    ],
)
def _emb_kernel(word_hbm, ids_hbm, tt_hbm, type_hbm, w_hbm, out_hbm,
                idx_v, tt_v, type_v, w_v, rows_v,
                sem0, sem1, sem2, sem3, sem_aux):
    wid = lax.axis_index("s") * 2 + lax.axis_index("c")
    brow = wid // WPB
    tok0 = (wid % WPB) * ROWS_PER_W
    sems = (sem0, sem1, sem2, sem3)

    pltpu.sync_copy(ids_hbm.at[brow, pl.ds(tok0, ROWS_PER_W)], idx_v)
    gathers = [
        pltpu.async_copy(
            word_hbm.at[idx_v.at[pl.ds(g * GCH, GCH)]],
            rows_v.at[pl.ds(g * GCH, GCH)],
            sems[g],
        )
        for g in range(NG)
    ]

    c_tt = pltpu.async_copy(tt_hbm.at[brow, pl.ds(tok0, ROWS_PER_W)], tt_v, sem_aux)
    c_ty = pltpu.async_copy(type_hbm, type_v, sem_aux)
    c_w = pltpu.async_copy(w_hbm, w_v, sem_aux)
    c_tt.wait()
    c_ty.wait()
    c_w.wait()

    t0 = [type_v[0, pl.ds(c * L, L)] for c in range(NCH)]
    td = [type_v[1, pl.ds(c * L, L)] - t0[c] for c in range(NCH)]
    wch = [w_v[pl.ds(c * L, L)] for c in range(NCH)]

    lanes = lax.iota(jnp.int32, L)
    perms = [lax.bitwise_xor(lanes, jnp.int32(k)) for k in (1, 2, 4, 8)]

    def group_body(g, carry):
        rbase = g * L
        ttf16 = tt_v[pl.ds(rbase, L)].astype(jnp.float32)
        for rr in range(L):
            r = rbase + rr
            ttf = jnp.broadcast_to(ttf16[rr], (L,))
            xs = []
            acc0 = jnp.zeros((L,), jnp.float32)
            acc1 = jnp.zeros((L,), jnp.float32)
            for c in range(NCH):
                xc = rows_v[r, pl.ds(c * L, L)] + (t0[c] + ttf * td[c])
                xs.append(xc)
                if c % 2 == 0:
                    acc0 = acc0 + xc * xc
                else:
                    acc1 = acc1 + xc * xc
            s = acc0 + acc1
            for p in perms:
                s = s + s.at[p].get(mode="promise_in_bounds")
            var = s * (1.0 / HIDDEN) + EPS
            scale = _rsqrt16(var)
            for c in range(NCH):
                rows_v[r, pl.ds(c * L, L)] = (xs[c] * scale) * wch[c]
        return carry

    GROUPS_PER_CHUNK = GCH // L
    stores = []
    for g in range(NG):
        gathers[g].wait()
        lax.fori_loop(g * GROUPS_PER_CHUNK, (g + 1) * GROUPS_PER_CHUNK,
                      group_body, 0)
        stores.append(pltpu.async_copy(
            rows_v.at[pl.ds(g * GCH, GCH)],
            out_hbm.at[brow, pl.ds(tok0 + g * GCH, GCH)],
            sems[g],
        ))
    for s in stores:
        s.wait()


def kernel(input_ids, token_type_ids, word_emb, type_emb, ln_weight):
    ids = input_ids.astype(jnp.int32)
    tt = token_type_ids.astype(jnp.int32)
    return _emb_kernel(word_emb, ids, tt, type_emb, ln_weight)


# speedup vs baseline: 1.3520x; 1.3520x over previous
import functools

import jax
import jax.numpy as jnp
from jax import lax
from jax.experimental import pallas as pl
from jax.experimental.pallas import tpu as pltpu
from jax.experimental.pallas import tpu_sc as plsc

HIDDEN = 128
B, T = 4, 2048
EPS = 1e-6
NTOK = B * T
NW = 32
ROWS_PER_W = NTOK // NW
WPB = T // ROWS_PER_W
L = 16
NCH = HIDDEN // L
GCH = 128
NG = ROWS_PER_W // GCH


def _rsqrt16(x):
    i = lax.bitcast_convert_type(x, jnp.int32)
    i = jnp.int32(0x5F3759DF) - lax.shift_right_arithmetic(i, 1)
    y = lax.bitcast_convert_type(i, jnp.float32)
    xh = x * 0.5
    for _ in range(3):
        y = y * (1.5 - xh * y * y)
    return y


@functools.partial(
    pl.kernel,
    out_type=jax.ShapeDtypeStruct((B, T, HIDDEN), jnp.float32),
    mesh=plsc.VectorSubcoreMesh(core_axis_name="c", subcore_axis_name="s"),
    scratch_types=[
        pltpu.VMEM((ROWS_PER_W,), jnp.int32),
        pltpu.VMEM((ROWS_PER_W,), jnp.int32),
        pltpu.VMEM((2, HIDDEN), jnp.float32),
        pltpu.VMEM((HIDDEN,), jnp.float32),
        pltpu.VMEM((ROWS_PER_W, HIDDEN), jnp.float32),
        pltpu.SemaphoreType.DMA,
        pltpu.SemaphoreType.DMA,
        pltpu.SemaphoreType.DMA,
    ],
)
def _emb_kernel(word_hbm, ids_hbm, tt_hbm, type_hbm, w_hbm, out_hbm,
                idx_v, tt_v, type_v, w_v, rows_v,
                sem0, sem1, sem_aux):
    wid = lax.axis_index("s") * 2 + lax.axis_index("c")
    brow = wid // WPB
    tok0 = (wid % WPB) * ROWS_PER_W
    sems = (sem0, sem1)

    pltpu.sync_copy(ids_hbm.at[brow, pl.ds(tok0, ROWS_PER_W)], idx_v)
    gathers = [
        pltpu.async_copy(
            word_hbm.at[idx_v.at[pl.ds(g * GCH, GCH)]],
            rows_v.at[pl.ds(g * GCH, GCH)],
            sems[g],
        )
        for g in range(NG)
    ]

    c_tt = pltpu.async_copy(tt_hbm.at[brow, pl.ds(tok0, ROWS_PER_W)], tt_v, sem_aux)
    c_ty = pltpu.async_copy(type_hbm, type_v, sem_aux)
    c_w = pltpu.async_copy(w_hbm, w_v, sem_aux)
    c_tt.wait()
    c_ty.wait()
    c_w.wait()

    t0 = [type_v[0, pl.ds(c * L, L)] for c in range(NCH)]
    td = [type_v[1, pl.ds(c * L, L)] - t0[c] for c in range(NCH)]
    wch = [w_v[pl.ds(c * L, L)] for c in range(NCH)]

    lanes = lax.iota(jnp.int32, L)
    perms = [lax.bitwise_xor(lanes, jnp.int32(k)) for k in (1, 2, 4, 8)]

    def group_body(g, carry):
        rbase = g * L
        ttf16 = tt_v[pl.ds(rbase, L)].astype(jnp.float32)
        for rr in range(L):
            r = rbase + rr
            ttf = jnp.broadcast_to(ttf16[rr], (L,))
            xs = []
            acc0 = jnp.zeros((L,), jnp.float32)
            acc1 = jnp.zeros((L,), jnp.float32)
            for c in range(NCH):
                xc = rows_v[r, pl.ds(c * L, L)] + (t0[c] + ttf * td[c])
                xs.append(xc)
                if c % 2 == 0:
                    acc0 = acc0 + xc * xc
                else:
                    acc1 = acc1 + xc * xc
            s = acc0 + acc1
            for p in perms:
                s = s + s.at[p].get(mode="promise_in_bounds")
            var = s * (1.0 / HIDDEN) + EPS
            scale = _rsqrt16(var)
            for c in range(NCH):
                rows_v[r, pl.ds(c * L, L)] = (xs[c] * scale) * wch[c]
        return carry

    for g in gathers:
        g.wait()
    lax.fori_loop(0, ROWS_PER_W // L, group_body, 0)
    pltpu.sync_copy(rows_v, out_hbm.at[brow, pl.ds(tok0, ROWS_PER_W)])


def kernel(input_ids, token_type_ids, word_emb, type_emb, ln_weight):
    ids = input_ids.astype(jnp.int32)
    tt = token_type_ids.astype(jnp.int32)
    return _emb_kernel(word_emb, ids, tt, type_emb, ln_weight)


# speedup vs baseline: 1.3830x; 1.0229x over previous
import functools

import jax
import jax.numpy as jnp
from jax import lax
from jax.experimental import pallas as pl
from jax.experimental.pallas import tpu as pltpu
from jax.experimental.pallas import tpu_sc as plsc

HIDDEN = 128
B, T = 4, 2048
EPS = 1e-6
NTOK = B * T
NW = 32
ROWS_PER_W = NTOK // NW
WPB = T // ROWS_PER_W
L = 16
NCH = HIDDEN // L
GCH = 128
NG = ROWS_PER_W // GCH


def _rsqrt16(x):
    i = lax.bitcast_convert_type(x, jnp.int32)
    i = jnp.int32(0x5F3759DF) - lax.shift_right_arithmetic(i, 1)
    y = lax.bitcast_convert_type(i, jnp.float32)
    xh = x * 0.5
    for _ in range(3):
        y = y * (1.5 - xh * y * y)
    return y


@functools.partial(
    pl.kernel,
    out_type=jax.ShapeDtypeStruct((B, T, HIDDEN), jnp.float32),
    mesh=plsc.VectorSubcoreMesh(core_axis_name="c", subcore_axis_name="s"),
    scratch_types=[
        pltpu.VMEM((ROWS_PER_W,), jnp.int32),
        pltpu.VMEM((ROWS_PER_W,), jnp.int32),
        pltpu.VMEM((2, HIDDEN), jnp.float32),
        pltpu.VMEM((HIDDEN,), jnp.float32),
        pltpu.VMEM((ROWS_PER_W, HIDDEN), jnp.float32),
        pltpu.SemaphoreType.DMA,
        pltpu.SemaphoreType.DMA,
        pltpu.SemaphoreType.DMA,
        pltpu.SemaphoreType.DMA,
    ],
)
def _emb_kernel(word_hbm, ids_hbm, tt_hbm, type_hbm, w_hbm, out_hbm,
                idx_v, tt_v, type_v, w_v, rows_v,
                sem0, sem1, sem_aux, sem_st):
    wid = lax.axis_index("s") * 2 + lax.axis_index("c")
    brow = wid // WPB
    tok0 = (wid % WPB) * ROWS_PER_W
    sems = (sem0, sem1)

    pltpu.sync_copy(ids_hbm.at[brow, pl.ds(tok0, ROWS_PER_W)], idx_v)
    gathers = [
        pltpu.async_copy(
            word_hbm.at[idx_v.at[pl.ds(g * GCH, GCH)]],
            rows_v.at[pl.ds(g * GCH, GCH)],
            sems[g],
        )
        for g in range(NG)
    ]

    c_tt = pltpu.async_copy(tt_hbm.at[brow, pl.ds(tok0, ROWS_PER_W)], tt_v, sem_aux)
    c_ty = pltpu.async_copy(type_hbm, type_v, sem_aux)
    c_w = pltpu.async_copy(w_hbm, w_v, sem_aux)
    c_tt.wait()
    c_ty.wait()
    c_w.wait()

    t0 = [type_v[0, pl.ds(c * L, L)] for c in range(NCH)]
    td = [type_v[1, pl.ds(c * L, L)] - t0[c] for c in range(NCH)]
    wch = [w_v[pl.ds(c * L, L)] for c in range(NCH)]

    lanes = lax.iota(jnp.int32, L)
    perms = [lax.bitwise_xor(lanes, jnp.int32(k)) for k in (1, 2, 4, 8)]

    QR = 64
    st_descs = [
        pltpu.make_async_copy(
            rows_v.at[pl.ds(q * QR, QR)],
            out_hbm.at[brow, pl.ds(tok0 + q * QR, QR)],
            sem_st,
        )
        for q in range(3)
    ]

    def group_body(g, carry):
        @pl.when(g == 8)
        def _wait_second_gather():
            gathers[1].wait()

        rbase = g * L
        ttf16 = tt_v[pl.ds(rbase, L)].astype(jnp.float32)
        for rr in range(L):
            r = rbase + rr
            ttf = jnp.broadcast_to(ttf16[rr], (L,))
            xs = []
            acc0 = jnp.zeros((L,), jnp.float32)
            acc1 = jnp.zeros((L,), jnp.float32)
            for c in range(NCH):
                xc = rows_v[r, pl.ds(c * L, L)] + (t0[c] + ttf * td[c])
                xs.append(xc)
                if c % 2 == 0:
                    acc0 = acc0 + xc * xc
                else:
                    acc1 = acc1 + xc * xc
            s = acc0 + acc1
            for p in perms:
                s = s + s.at[p].get(mode="promise_in_bounds")
            var = s * (1.0 / HIDDEN) + EPS
            scale = _rsqrt16(var)
            for c in range(NCH):
                rows_v[r, pl.ds(c * L, L)] = (xs[c] * scale) * wch[c]

        for q in range(3):
            @pl.when(g == 4 * q + 3)
            def _store_quarter(q=q):
                st_descs[q].start()

        return carry

    gathers[0].wait()
    lax.fori_loop(0, ROWS_PER_W // L, group_body, 0)
    pltpu.sync_copy(rows_v.at[pl.ds(3 * QR, QR)],
                    out_hbm.at[brow, pl.ds(tok0 + 3 * QR, QR)])
    for q in range(3):
        st_descs[q].wait()


def kernel(input_ids, token_type_ids, word_emb, type_emb, ln_weight):
    ids = input_ids.astype(jnp.int32)
    tt = token_type_ids.astype(jnp.int32)
    return _emb_kernel(word_emb, ids, tt, type_emb, ln_weight)
